# Initial kernel scaffold; baseline (speedup 1.0000x reference)
import functools

import jax
import jax.numpy as jnp
from jax import lax
from jax.experimental import pallas as pl
from jax.experimental.pallas import tpu as pltpu
from jax.experimental.pallas import tpu_sc as plsc

NX = 512
NY = 512
NCH = 64
NVOX = 20000

NC = 2
NS = 16
NW = NC * NS
RANGE = (NX * NY) // NW
RBITS = 13

CPASS = 8
NPASS = NCH // CPASS
CHUNK = 2000
NCHUNK = NVOX // CHUNK
GPC = CHUNK // 16
K = 512
OWNCAP = NVOX + 64


def _nextof(x, iota):
    return jnp.take(x, jnp.minimum(iota + 1, 15), mode="promise_in_bounds")


def _body(vf8, c0_hbm, c1_hbm, out_hbm, slab, owned, c0b, c1b, gidx, featb, sem):
    wid = lax.axis_index("s") * NC + lax.axis_index("c")
    lo = wid * RANGE
    iota = lax.broadcasted_iota(jnp.int32, (16,), 0)

    def chunk_body(g, cursor):
        pltpu.sync_copy(c0_hbm.at[pl.ds(g * CHUNK, CHUNK)], c0b)
        pltpu.sync_copy(c1_hbm.at[pl.ds(g * CHUNK, CHUNK)], c1b)

        def grp(i, cur):
            v0 = c0b[pl.ds(i * 16, 16)]
            v1 = c1b[pl.ds(i * 16, 16)]
            local = (v0 + v1 * NX) - lo
            m = (local >= 0) & (local < RANGE)
            vid = g * CHUNK + i * 16 + iota
            packed = (vid << RBITS) | jnp.where(m, local, 0)
            plsc.store_compressed(owned.at[pl.ds(cur, 16)], packed, m)
            return cur + jnp.sum(m.astype(jnp.int32))

        return lax.fori_loop(0, GPC, grp, cursor)

    n = lax.fori_loop(0, NCHUNK, chunk_body, jnp.int32(0))
    nchunks = (n + K - 1) // K

    for p in range(NPASS):
        def zrow(i, _):
            slab[pl.ds(i * 16, 16)] = jnp.zeros((16,), jnp.float32)
            return 0

        lax.fori_loop(0, (CPASS * RANGE) // 16, zrow, 0)

        def kchunk(k, _):
            base = k * K

            def gi(j, _):
                pk = owned[pl.ds(base + j * 16, 16)]
                ok = (base + j * 16 + iota) < n
                vid = jnp.minimum(pk >> RBITS, NVOX - 1)
                gidx[pl.ds(j * 16, 16)] = jnp.where(
                    ok, vid * NPASS + p, j * 16 + iota)
                return 0

            lax.fori_loop(0, K // 16, gi, 0)
            pltpu.async_copy(vf8.at[gidx], featb, sem).wait()

            def sc(j, _):
                pk = owned[pl.ds(base + j * 16, 16)]
                local = pk & (RANGE - 1)
                valid = (base + j * 16 + iota) < n
                key2 = (jnp.where(valid, local, RANGE + iota) << 4) | iota
                sk, sv = plsc.sort_key_val(key2, iota)
                skey = sk >> 4
                is_last = (_nextof(skey, iota) != skey) | (iota == 15)
                m = is_last & (skey < RANGE)
                for c in range(CPASS):
                    cvec = jnp.full((16,), c, jnp.int32)
                    vals = plsc.load_gather(featb, [j * 16 + sv, cvec])
                    plsc.store_scatter(slab, [(cvec << RBITS) | skey],
                                       vals, mask=m)
                return 0

            lax.fori_loop(0, K // 16, sc, 0)
            return 0

        lax.fori_loop(0, nchunks, kchunk, 0)

        slab2d = slab.reshape(CPASS, RANGE)
        pltpu.sync_copy(
            slab2d, out_hbm.at[pl.ds(p * CPASS, CPASS), pl.ds(lo, RANGE)])


@jax.jit
def kernel(voxel_features, coords):
    coords = coords.astype(jnp.int32)
    c0 = coords[:, 0]
    c1 = coords[:, 1]
    vf8 = voxel_features.reshape(NVOX * NPASS, CPASS)

    mesh = plsc.VectorSubcoreMesh(core_axis_name="c", subcore_axis_name="s")
    run = functools.partial(
        pl.kernel,
        out_type=jax.ShapeDtypeStruct((NCH, NX * NY), jnp.float32),
        mesh=mesh,
        scratch_types=[
            pltpu.VMEM((CPASS * RANGE,), jnp.float32),
            pltpu.VMEM((OWNCAP,), jnp.int32),
            pltpu.VMEM((CHUNK,), jnp.int32),
            pltpu.VMEM((CHUNK,), jnp.int32),
            pltpu.VMEM((K,), jnp.int32),
            pltpu.VMEM((K, CPASS), jnp.float32),
            pltpu.SemaphoreType.DMA,
        ],
    )(_body)
    canvas = run(vf8, c0, c1)
    return canvas.reshape(NCH, NX, NY)


# speedup vs baseline: 1.3974x; 1.3974x over previous
import functools

import jax
import jax.numpy as jnp
from jax import lax
from jax.experimental import pallas as pl
from jax.experimental.pallas import tpu as pltpu
from jax.experimental.pallas import tpu_sc as plsc

NX = 512
NY = 512
NCH = 64
NVOX = 20000

NC = 2
NS = 16
NW = NC * NS
RANGE = (NX * NY) // NW
RBITS = 13

CPASS = 8
NPASS = NCH // CPASS
CHUNK = 2000
NCHUNK = NVOX // CHUNK
GPC = CHUNK // 16
K = 128
OWNCAP = NVOX + 2 * K


_GDN = lax.GatherDimensionNumbers(
    offset_dims=(), collapsed_slice_dims=(0,), start_index_map=(0,))


def _nextof(x, iota):
    idx = jnp.minimum(iota + 1, 15)
    return lax.gather(x, idx[:, None], _GDN, slice_sizes=(1,),
                      mode=lax.GatherScatterMode.PROMISE_IN_BOUNDS)


def _body(vf8, c0_hbm, c1_hbm, out_hbm, slab, owned, c0b, c1b, gidx, featb, sem):
    wid = lax.axis_index("s") * NC + lax.axis_index("c")
    lo = wid * RANGE
    iota = lax.broadcasted_iota(jnp.int32, (16,), 0)

    def chunk_body(g, cursor):
        pltpu.sync_copy(c0_hbm.at[pl.ds(g * CHUNK, CHUNK)], c0b)
        pltpu.sync_copy(c1_hbm.at[pl.ds(g * CHUNK, CHUNK)], c1b)

        def grp(i, cur):
            v0 = c0b[pl.ds(i * 16, 16)]
            v1 = c1b[pl.ds(i * 16, 16)]
            local = (v0 + v1 * NX) - lo
            m = (local >= 0) & (local < RANGE)
            vid = g * CHUNK + i * 16 + iota
            packed = (vid << RBITS) | jnp.where(m, local, 0)
            mi = m.astype(jnp.int32)
            pos = cur + plsc.cumsum(mi) - 1
            plsc.store_scatter(owned, [pos], packed, mask=m)
            return cur + jnp.sum(mi)

        return lax.fori_loop(0, GPC, grp, cursor)

    n = lax.fori_loop(0, NCHUNK, chunk_body, jnp.int32(0))
    nchunks = (n + K - 1) // K

    for p in range(NPASS):
        def zrow(i, _):
            for c in range(CPASS):
                slab[c, pl.ds(i * 16, 16)] = jnp.zeros((16,), jnp.float32)
            return 0

        lax.fori_loop(0, RANGE // 16, zrow, 0)

        def kchunk(k, _):
            base = k * K

            def gi(j, _):
                pk = owned[pl.ds(base + j * 16, 16)]
                ok = (base + j * 16 + iota) < n
                vid = jnp.minimum(pk >> RBITS, NVOX - 1)
                gidx[pl.ds(j * 16, 16)] = jnp.where(
                    ok, vid * NPASS + p, j * 16 + iota)
                return 0

            lax.fori_loop(0, K // 16, gi, 0)
            pltpu.async_copy(vf8.at[gidx], featb, sem).wait()

            def sc(j, _):
                pk = owned[pl.ds(base + j * 16, 16)]
                local = pk & (RANGE - 1)
                valid = (base + j * 16 + iota) < n
                key2 = (jnp.where(valid, local, RANGE + iota) << 4) | iota
                sk, sv = plsc.sort_key_val(key2, iota)
                skey = sk >> 4
                is_last = (_nextof(skey, iota) != skey) | (iota == 15)
                m = is_last & (skey < RANGE)
                for c in range(CPASS):
                    cvec = jnp.full((16,), c, jnp.int32)
                    vals = plsc.load_gather(featb, [j * 16 + sv, cvec])
                    plsc.store_scatter(slab, [cvec, skey], vals, mask=m)
                return 0

            lax.fori_loop(0, K // 16, sc, 0)
            return 0

        lax.fori_loop(0, nchunks, kchunk, 0)

        pltpu.sync_copy(
            slab, out_hbm.at[pl.ds(p * CPASS, CPASS), pl.ds(lo, RANGE)])


@jax.jit
def kernel(voxel_features, coords):
    coords = coords.astype(jnp.int32)
    c0 = coords[:, 0]
    c1 = coords[:, 1]
    vf8 = voxel_features.reshape(NVOX * NPASS, CPASS)

    mesh = plsc.VectorSubcoreMesh(core_axis_name="c", subcore_axis_name="s")
    run = functools.partial(
        pl.kernel,
        out_type=jax.ShapeDtypeStruct((NCH, NX * NY), jnp.float32),
        mesh=mesh,
        compiler_params=pltpu.CompilerParams(
            needs_layout_passes=False, use_tc_tiling_on_sc=False),
        scratch_types=[
            pltpu.VMEM((CPASS, RANGE), jnp.float32),
            pltpu.VMEM((OWNCAP,), jnp.int32),
            pltpu.VMEM((CHUNK,), jnp.int32),
            pltpu.VMEM((CHUNK,), jnp.int32),
            pltpu.VMEM((K,), jnp.int32),
            pltpu.VMEM((K, CPASS), jnp.float32),
            pltpu.SemaphoreType.DMA,
        ],
    )(_body)
    canvas = run(vf8, c0, c1)
    return canvas.reshape(NCH, NX, NY)
